# baseline (speedup 1.0000x reference)
import jax
import jax.numpy as jnp
from jax.experimental import pallas as pl


def kernel(memory, nids, val):
    B = nids.shape[0]
    N = memory.shape[0]
    order = jnp.arange(B, dtype=jnp.int32)
    winner = jnp.zeros((N,), jnp.int32).at[nids].max(order)
    keep = winner[nids] == order
    safe = jnp.where(keep, nids, N)
    return memory.at[safe].set(val, mode="drop")


# speedup vs baseline: 2.1861x; 2.1861x over previous
import functools

import jax
import jax.numpy as jnp
from jax import lax
from jax.experimental import pallas as pl
from jax.experimental.pallas import tpu as pltpu
from jax.experimental.pallas import tpu_sc as plsc

N_NODES = 1000000
DIM = 16
BATCH = 16384
L = 16

NC = 2
NS = 16
NW = NC * NS
ROWS_PER_W = N_NODES // NW
T_SIZE = ((ROWS_PER_W + L - 1) // L) * L
CHUNK = 128

_mesh = plsc.VectorSubcoreMesh(core_axis_name="c", subcore_axis_name="s")


@functools.partial(
    pl.kernel,
    mesh=_mesh,
    compiler_params=pltpu.CompilerParams(
        needs_layout_passes=False, use_tc_tiling_on_sc=False),
    scratch_types=[
        pltpu.VMEM((BATCH,), jnp.int32),
        pltpu.VMEM((T_SIZE,), jnp.int32),
        pltpu.VMEM((BATCH,), jnp.int32),
        pltpu.VMEM((BATCH,), jnp.int32),
        pltpu.VMEM((CHUNK,), jnp.int32),
        pltpu.VMEM((CHUNK,), jnp.int32),
        pltpu.VMEM((CHUNK, DIM), jnp.float32),
        pltpu.SemaphoreType.DMA,
        pltpu.SemaphoreType.DMA,
    ],
)
def _sc_scatter(nids_hbm, val_hbm, out_hbm,
                nids_v, t_v, wb_v, wn_v, idxb_c, idxn_c, rows_v,
                sem_g, sem_s):
    wid = lax.axis_index("s") * NC + lax.axis_index("c")
    base = wid * ROWS_PER_W
    iota = lax.iota(jnp.int32, L)
    neg1 = jnp.full((L,), -1, jnp.int32)

    pltpu.sync_copy(nids_hbm, nids_v)

    def init_body(i, carry):
        t_v[pl.ds(i * L, L)] = neg1
        return carry
    lax.fori_loop(0, T_SIZE // L, init_body, 0, unroll=4)

    def stamp_body(i, carry):
        v = nids_v[pl.ds(i * L, L)]
        inr = (v >= base) & (v < base + ROWS_PER_W)
        _, last = plsc.scan_count(v, mask=inr)
        m = inr & last
        local = jnp.where(m, v - base, 0)
        bidx = iota + i * L
        plsc.store_scatter(t_v, [local], bidx, mask=m)
        return carry
    lax.fori_loop(0, BATCH // L, stamp_body, 0, unroll=2)

    def compact_body(k, cnt):
        t = t_v[pl.ds(k * L, L)]
        m = t >= 0
        m_i32 = m.astype(jnp.int32)
        inc = plsc.cumsum(m_i32)
        pos = cnt + inc - m_i32
        nvec = base + k * L + iota
        plsc.store_scatter(wb_v, [pos], t, mask=m)
        plsc.store_scatter(wn_v, [pos], nvec, mask=m)
        return cnt + jnp.max(inc)
    cnt = lax.fori_loop(0, T_SIZE // L, compact_body, jnp.int32(0), unroll=2)

    @pl.when(cnt > 0)
    def _tail():
        nchunks = (cnt + CHUNK - 1) // CHUNK
        cnt_pad = nchunks * CHUNK
        aligned = (cnt - 1) & ~(L - 1)
        vb = wb_v[pl.ds(aligned, L)]
        vn = wn_v[pl.ds(aligned, L)]
        lane = cnt - 1 - aligned
        b_last = jnp.max(jnp.where(iota == lane, vb, jnp.int32(-2147483648)))
        n_last = jnp.max(jnp.where(iota == lane, vn, jnp.int32(-2147483648)))
        b_splat = jnp.full((L,), 0, jnp.int32) + b_last
        n_splat = jnp.full((L,), 0, jnp.int32) + n_last

        def pad_body(p, carry):
            pvec = p * L + iota
            fm = pvec >= cnt
            plsc.store_scatter(wb_v, [pvec], b_splat, mask=fm)
            plsc.store_scatter(wn_v, [pvec], n_splat, mask=fm)
            return carry
        lax.fori_loop(aligned // L, cnt_pad // L, pad_body, 0)

        def chunk_body(c, carry):
            for j in range(CHUNK // L):
                off = c * CHUNK + j * L
                idxb_c[pl.ds(j * L, L)] = wb_v[pl.ds(off, L)]
                idxn_c[pl.ds(j * L, L)] = wn_v[pl.ds(off, L)]
            pltpu.async_copy(val_hbm.at[idxb_c], rows_v, sem_g).wait()
            pltpu.async_copy(rows_v, out_hbm.at[idxn_c], sem_s).wait()
            return carry
        lax.fori_loop(0, nchunks, chunk_body, 0)


def kernel(memory, nids, val):
    out = jax.new_ref(memory)
    _sc_scatter(nids, val, out)
    return out[...]
